# baseline (speedup 1.0000x reference)
import functools

import jax
import jax.numpy as jnp
from jax import lax
from jax.experimental import pallas as pl
from jax.experimental.pallas import tpu as pltpu

_VMEM_LIMIT = 56 * 1024 * 1024


def _cell(gates, c_prev, H):
    sig_if = jax.nn.sigmoid(gates[:, 0 * H:2 * H])
    i = sig_if[:, :H]
    f = sig_if[:, H:]
    g = jnp.tanh(gates[:, 2 * H:3 * H])
    o = jax.nn.sigmoid(gates[:, 3 * H:4 * H])
    c_new = f * c_prev + i * g
    return o * jnp.tanh(c_new), c_new


def _encoder_kernel(ids_ref, p0f_ref, p0b_ref, whh0f_ref, whh0b_ref, w1_ref,
                    b1_ref, whh1f_ref, whh1b_ref, pooled_ref,
                    hf_ref, hb_ref, cf_ref, cb_ref, out01_ref, gx1_ref,
                    gxf_buf, gxb_buf, semf, semb,
                    *, chunk, hidden, seq_len, n_batch):
    Tt, H, T = chunk, hidden, seq_len
    H4 = 4 * H
    Nc = T // Tt
    Bc = ids_ref.shape[1] // n_batch
    rows = Tt * Bc
    i = pl.program_id(0)
    c = pl.program_id(1)
    slot = lax.rem(c, 2)
    nslot = 1 - slot
    col0 = i * Bc

    def issue_row(t_f, t_b, b, sl, s):
        idf = ids_ref[t_f, col0 + b]
        idb = ids_ref[t_b, col0 + b]
        pltpu.make_async_copy(p0f_ref.at[idf], gxf_buf.at[sl, s, b],
                              semf.at[sl]).start()
        pltpu.make_async_copy(p0b_ref.at[idb], gxb_buf.at[sl, s, b],
                              semb.at[sl]).start()

    def wait_chunk(sl):
        pltpu.make_async_copy(p0f_ref.at[pl.ds(0, rows)],
                              gxf_buf.at[sl].reshape(rows, H4),
                              semf.at[sl]).wait()
        pltpu.make_async_copy(p0b_ref.at[pl.ds(0, rows)],
                              gxb_buf.at[sl].reshape(rows, H4),
                              semb.at[sl]).wait()

    @pl.when(c == 0)
    def _cold():
        for r in (hf_ref, hb_ref, cf_ref, cb_ref):
            r[...] = jnp.zeros_like(r)

        def body(s, carry):
            for b in range(Bc):
                issue_row(s, (Nc - 1) * Tt + s, b, 0, s)
            return carry

        lax.fori_loop(0, Tt, body, 0)

    wait_chunk(slot)

    cn = lax.rem(c + 1, Nc)
    tfn = cn * Tt
    tbn = (Nc - 1 - cn) * Tt

    whh_f = whh0f_ref[...]
    whh_b = whh0b_ref[...]
    h_f, h_b = hf_ref[...], hb_ref[...]
    c_f, c_b = cf_ref[...], cb_ref[...]
    t0f = c * Tt
    t0b = (Nc - 1 - c) * Tt
    for s in range(Tt):
        for b in range(Bc):
            issue_row(tfn + s, tbn + s, b, nslot, s)
        gxf = gxf_buf[slot, s]
        gxb = gxb_buf[slot, Tt - 1 - s]
        g_f = gxf + jnp.dot(h_f, whh_f, preferred_element_type=jnp.float32)
        g_b = gxb + jnp.dot(h_b, whh_b, preferred_element_type=jnp.float32)
        h_f, c_f = _cell(g_f, c_f, H)
        h_b, c_b = _cell(g_b, c_b, H)
        out01_ref[t0f + s, :, :H] = h_f
        out01_ref[t0b + (Tt - 1 - s), :, H:] = h_b
    hf_ref[...], hb_ref[...] = h_f, h_b
    cf_ref[...], cb_ref[...] = c_f, c_b

    @pl.when(c >= (Nc + 1) // 2)
    def _proj_pair():
        w1 = w1_ref[...]
        b1 = b1_ref[...]
        for k in (c, Nc - 1 - c):
            x = out01_ref[pl.ds(k * Tt, Tt)].reshape(rows, 2 * H)
            gx1_ref[pl.ds(k * Tt, Tt)] = (
                jnp.dot(x, w1, preferred_element_type=jnp.float32) + b1
            ).reshape(Tt, Bc, 8 * H)

    @pl.when(c == Nc - 1)
    def _layer1():
        if Nc % 2 == 1:
            k = Nc // 2
            x = out01_ref[pl.ds(k * Tt, Tt)].reshape(rows, 2 * H)
            gx1_ref[pl.ds(k * Tt, Tt)] = (
                jnp.dot(x, w1_ref[...], preferred_element_type=jnp.float32)
                + b1_ref[...]
            ).reshape(Tt, Bc, 8 * H)

        whh1_f = whh1f_ref[...]
        whh1_b = whh1b_ref[...]
        zeros = jnp.zeros((Bc, H), jnp.float32)

        def body(k, carry):
            h1f, c1f, h1b, c1b, a_f, a_b = carry
            for u in range(Tt):
                s = k * Tt + u
                g_f = gx1_ref[s, :, :H4] + jnp.dot(
                    h1f, whh1_f, preferred_element_type=jnp.float32)
                g_b = gx1_ref[T - 1 - s, :, H4:] + jnp.dot(
                    h1b, whh1_b, preferred_element_type=jnp.float32)
                h1f, c1f = _cell(g_f, c1f, H)
                h1b, c1b = _cell(g_b, c1b, H)
                a_f = a_f + h1f
                a_b = a_b + h1b
            return (h1f, c1f, h1b, c1b, a_f, a_b)

        carry = (zeros, zeros, zeros, zeros, zeros, zeros)
        _, _, _, _, a_f, a_b = lax.fori_loop(0, Nc, body, carry)
        inv_t = jnp.float32(1.0 / T)
        pooled_ref[:, :H] = a_f * inv_t
        pooled_ref[:, H:] = a_b * inv_t
        wait_chunk(nslot)


def kernel(p0_f, p0_b, whh0_f_t, whh0_b_t, w1_top, w1_bot, b1, whh1_f_t, whh1_b_t, ids):
    B, T = ids.shape
    H = whh0_f_t.shape[0]
    H4 = 4 * H

    ids_tm = ids.T
    w1 = jnp.concatenate([w1_top, w1_bot], axis=0)

    chunk = 4
    Nc = T // chunk
    NB = 2
    Bc = B // NB

    body = functools.partial(_encoder_kernel, chunk=chunk, hidden=H,
                             seq_len=T, n_batch=NB)
    grid_spec = pltpu.PrefetchScalarGridSpec(
        num_scalar_prefetch=1,
        grid=(NB, Nc),
        in_specs=[
            pl.BlockSpec(memory_space=pl.ANY),
            pl.BlockSpec(memory_space=pl.ANY),
            pl.BlockSpec((H, H4), lambda i, c, ids: (0, 0)),
            pl.BlockSpec((H, H4), lambda i, c, ids: (0, 0)),
            pl.BlockSpec((2 * H, 8 * H), lambda i, c, ids: (0, 0)),
            pl.BlockSpec((1, 8 * H), lambda i, c, ids: (0, 0)),
            pl.BlockSpec((H, H4), lambda i, c, ids: (0, 0)),
            pl.BlockSpec((H, H4), lambda i, c, ids: (0, 0)),
        ],
        out_specs=pl.BlockSpec((Bc, 2 * H), lambda i, c, ids: (i, 0)),
        scratch_shapes=[pltpu.VMEM((Bc, H), jnp.float32) for _ in range(4)]
                       + [pltpu.VMEM((T, Bc, 2 * H), jnp.float32),
                          pltpu.VMEM((T, Bc, 8 * H), jnp.float32),
                          pltpu.VMEM((2, chunk, Bc, H4), jnp.float32),
                          pltpu.VMEM((2, chunk, Bc, H4), jnp.float32),
                          pltpu.SemaphoreType.DMA((2,)),
                          pltpu.SemaphoreType.DMA((2,))],
    )
    pooled = pl.pallas_call(
        body,
        out_shape=jax.ShapeDtypeStruct((B, 2 * H), jnp.float32),
        grid_spec=grid_spec,
        compiler_params=pltpu.CompilerParams(
            dimension_semantics=("parallel", "arbitrary"),
            vmem_limit_bytes=_VMEM_LIMIT),
    )(ids_tm, p0_f, p0_b, whh0_f_t, whh0_b_t, w1, b1, whh1_f_t, whh1_b_t)
    return pooled


# speedup vs baseline: 1.1115x; 1.1115x over previous
import functools

import jax
import jax.numpy as jnp
from jax import lax
from jax.experimental import pallas as pl
from jax.experimental.pallas import tpu as pltpu

_VMEM_LIMIT = 56 * 1024 * 1024


def _cell(gates, c_prev, H):
    sig_if = jax.nn.sigmoid(gates[:, 0 * H:2 * H])
    i = sig_if[:, :H]
    f = sig_if[:, H:]
    g = jnp.tanh(gates[:, 2 * H:3 * H])
    o = jax.nn.sigmoid(gates[:, 3 * H:4 * H])
    c_new = f * c_prev + i * g
    return o * jnp.tanh(c_new), c_new


def _encoder_kernel(ids_ref, p0f_ref, p0b_ref, whh0f_ref, whh0b_ref, w1_ref,
                    b1_ref, whh1f_ref, whh1b_ref, pooled_ref,
                    hf_ref, hb_ref, cf_ref, cb_ref, out01_ref, gx1_ref,
                    gxf_buf, gxb_buf, semf, semb,
                    *, chunk, hidden, seq_len, n_batch):
    Tt, H, T = chunk, hidden, seq_len
    H4 = 4 * H
    Nc = T // Tt
    Bc = ids_ref.shape[1] // n_batch
    rows = Tt * Bc
    i = pl.program_id(0)
    c = pl.program_id(1)
    slot = lax.rem(c, 2)
    nslot = 1 - slot
    col0 = i * Bc

    def issue_row(t_f, t_b, b, sl, s):
        idf = ids_ref[t_f, col0 + b]
        idb = ids_ref[t_b, col0 + b]
        pltpu.make_async_copy(p0f_ref.at[idf], gxf_buf.at[sl, s, b],
                              semf.at[sl]).start()
        pltpu.make_async_copy(p0b_ref.at[idb], gxb_buf.at[sl, s, b],
                              semb.at[sl]).start()

    def wait_chunk(sl):
        pltpu.make_async_copy(p0f_ref.at[pl.ds(0, rows)],
                              gxf_buf.at[sl].reshape(rows, H4),
                              semf.at[sl]).wait()
        pltpu.make_async_copy(p0b_ref.at[pl.ds(0, rows)],
                              gxb_buf.at[sl].reshape(rows, H4),
                              semb.at[sl]).wait()

    @pl.when(c == 0)
    def _cold():
        for r in (hf_ref, hb_ref, cf_ref, cb_ref):
            r[...] = jnp.zeros_like(r)

        def body(s, carry):
            for b in range(Bc):
                issue_row(s, (Nc - 1) * Tt + s, b, 0, s)
            return carry

        lax.fori_loop(0, Tt, body, 0)

    wait_chunk(slot)

    cn = lax.rem(c + 1, Nc)
    tfn = cn * Tt
    tbn = (Nc - 1 - cn) * Tt

    whh_f = whh0f_ref[...]
    whh_b = whh0b_ref[...]
    h_f, h_b = hf_ref[...], hb_ref[...]
    c_f, c_b = cf_ref[...], cb_ref[...]
    t0f = c * Tt
    t0b = (Nc - 1 - c) * Tt
    for s in range(Tt):
        for b in range(Bc):
            issue_row(tfn + s, tbn + s, b, nslot, s)
        gxf = gxf_buf[slot, s]
        gxb = gxb_buf[slot, Tt - 1 - s]
        g_f = gxf + jnp.dot(h_f, whh_f, preferred_element_type=jnp.float32)
        g_b = gxb + jnp.dot(h_b, whh_b, preferred_element_type=jnp.float32)
        h_f, c_f = _cell(g_f, c_f, H)
        h_b, c_b = _cell(g_b, c_b, H)
        out01_ref[t0f + s, :, :H] = h_f
        out01_ref[t0b + (Tt - 1 - s), :, H:] = h_b
    hf_ref[...], hb_ref[...] = h_f, h_b
    cf_ref[...], cb_ref[...] = c_f, c_b

    @pl.when(c >= (Nc + 1) // 2)
    def _proj_pair():
        w1 = w1_ref[...]
        b1 = b1_ref[...]
        for k in (c, Nc - 1 - c):
            x = out01_ref[pl.ds(k * Tt, Tt)].reshape(rows, 2 * H)
            gx1_ref[pl.ds(k * Tt, Tt)] = (
                jnp.dot(x, w1, preferred_element_type=jnp.float32) + b1
            ).reshape(Tt, Bc, 8 * H)

    @pl.when(c == Nc - 1)
    def _layer1():
        if Nc % 2 == 1:
            k = Nc // 2
            x = out01_ref[pl.ds(k * Tt, Tt)].reshape(rows, 2 * H)
            gx1_ref[pl.ds(k * Tt, Tt)] = (
                jnp.dot(x, w1_ref[...], preferred_element_type=jnp.float32)
                + b1_ref[...]
            ).reshape(Tt, Bc, 8 * H)

        whh1_f = whh1f_ref[...]
        whh1_b = whh1b_ref[...]
        zeros = jnp.zeros((Bc, H), jnp.float32)

        def body(k, carry):
            h1f, c1f, h1b, c1b, a_f, a_b = carry
            for u in range(Tt):
                s = k * Tt + u
                g_f = gx1_ref[s, :, :H4] + jnp.dot(
                    h1f, whh1_f, preferred_element_type=jnp.float32)
                g_b = gx1_ref[T - 1 - s, :, H4:] + jnp.dot(
                    h1b, whh1_b, preferred_element_type=jnp.float32)
                h1f, c1f = _cell(g_f, c1f, H)
                h1b, c1b = _cell(g_b, c1b, H)
                a_f = a_f + h1f
                a_b = a_b + h1b
            return (h1f, c1f, h1b, c1b, a_f, a_b)

        carry = (zeros, zeros, zeros, zeros, zeros, zeros)
        _, _, _, _, a_f, a_b = lax.fori_loop(0, Nc, body, carry)
        inv_t = jnp.float32(1.0 / T)
        pooled_ref[:, :H] = a_f * inv_t
        pooled_ref[:, H:] = a_b * inv_t
        wait_chunk(nslot)


def kernel(p0_f, p0_b, whh0_f_t, whh0_b_t, w1_top, w1_bot, b1, whh1_f_t, whh1_b_t, ids):
    B, T = ids.shape
    H = whh0_f_t.shape[0]
    H4 = 4 * H

    ids_tm = ids.T
    w1 = jnp.concatenate([w1_top, w1_bot], axis=0)

    chunk = 8
    Nc = T // chunk
    NB = 2
    Bc = B // NB

    body = functools.partial(_encoder_kernel, chunk=chunk, hidden=H,
                             seq_len=T, n_batch=NB)
    grid_spec = pltpu.PrefetchScalarGridSpec(
        num_scalar_prefetch=1,
        grid=(NB, Nc),
        in_specs=[
            pl.BlockSpec(memory_space=pl.ANY),
            pl.BlockSpec(memory_space=pl.ANY),
            pl.BlockSpec((H, H4), lambda i, c, ids: (0, 0)),
            pl.BlockSpec((H, H4), lambda i, c, ids: (0, 0)),
            pl.BlockSpec((2 * H, 8 * H), lambda i, c, ids: (0, 0)),
            pl.BlockSpec((1, 8 * H), lambda i, c, ids: (0, 0)),
            pl.BlockSpec((H, H4), lambda i, c, ids: (0, 0)),
            pl.BlockSpec((H, H4), lambda i, c, ids: (0, 0)),
        ],
        out_specs=pl.BlockSpec((Bc, 2 * H), lambda i, c, ids: (i, 0)),
        scratch_shapes=[pltpu.VMEM((Bc, H), jnp.float32) for _ in range(4)]
                       + [pltpu.VMEM((T, Bc, 2 * H), jnp.float32),
                          pltpu.VMEM((T, Bc, 8 * H), jnp.float32),
                          pltpu.VMEM((2, chunk, Bc, H4), jnp.float32),
                          pltpu.VMEM((2, chunk, Bc, H4), jnp.float32),
                          pltpu.SemaphoreType.DMA((2,)),
                          pltpu.SemaphoreType.DMA((2,))],
    )
    pooled = pl.pallas_call(
        body,
        out_shape=jax.ShapeDtypeStruct((B, 2 * H), jnp.float32),
        grid_spec=grid_spec,
        compiler_params=pltpu.CompilerParams(
            dimension_semantics=("parallel", "arbitrary"),
            vmem_limit_bytes=_VMEM_LIMIT),
    )(ids_tm, p0_f, p0_b, whh0_f_t, whh0_b_t, w1, b1, whh1_f_t, whh1_b_t)
    return pooled
